# baseline (speedup 1.0000x reference)
import functools

import jax
import jax.numpy as jnp
from jax import lax
from jax.experimental import pallas as pl
from jax.experimental.pallas import tpu as pltpu
from jax.experimental.pallas import tpu_sc as plsc

B = 4
M = 8192
D = 768
SB = 2
NC = 2
NS = 16
NW = NC * NS
POS_PER_W = M // NW
CH = 32
CHUNKS = POS_PER_W // CH
VPR = D // 16
NIT = CHUNKS * SB


def _pos_add_body(x_hbm, pos_hbm, out_hbm,
                  xv0, xv1, pv0, pv1,
                  sin0, sin1, sout0, sout1, sp0, sp1):
    wid = lax.axis_index("s") * NC + lax.axis_index("c")
    row0 = wid * POS_PER_W
    xv = [xv0, xv1]
    pv = [pv0, pv1]
    sin = [sin0, sin1]
    sout = [sout0, sout1]
    sp = [sp0, sp1]

    def x_row(g):
        c, b = divmod(g, SB)
        return b * M + row0 + c * CH

    in_h = [None] * NIT
    out_h = [None] * NIT
    pos_h = [None] * CHUNKS

    pos_h[0] = pltpu.async_copy(pos_hbm.at[pl.ds(row0, CH)], pv[0], sp[0])
    in_h[0] = pltpu.async_copy(x_hbm.at[pl.ds(x_row(0), CH)], xv[0], sin[0])

    for g in range(NIT):
        s = g % 2
        c = g // SB
        if g % SB == 0 and c + 1 < CHUNKS:
            pos_h[c + 1] = pltpu.async_copy(
                pos_hbm.at[pl.ds(row0 + (c + 1) * CH, CH)],
                pv[(c + 1) % 2], sp[(c + 1) % 2])
        if g + 1 < NIT:
            if g >= 1:
                out_h[g - 1].wait()
            in_h[g + 1] = pltpu.async_copy(
                x_hbm.at[pl.ds(x_row(g + 1), CH)], xv[1 - s], sin[1 - s])
        if g % SB == 0:
            pos_h[c].wait()
        in_h[g].wait()

        pvs = pv[c % 2]
        xvs = xv[s]

        @plsc.parallel_loop(0, CH, unroll=1)
        def _row_body(r):
            @plsc.parallel_loop(0, VPR, unroll=16)
            def _vec_body(v):
                j = v * 16
                plsc.addupdate(xvs.at[r, pl.ds(j, 16)], pvs[r, pl.ds(j, 16)])

        out_h[g] = pltpu.async_copy(xvs, out_hbm.at[pl.ds(x_row(g), CH)], sout[s])

    out_h[NIT - 2].wait()
    out_h[NIT - 1].wait()


_pos_add_sc = functools.partial(
    pl.kernel,
    out_type=jax.ShapeDtypeStruct((SB * M, D), jnp.float32),
    mesh=plsc.VectorSubcoreMesh(core_axis_name="c", subcore_axis_name="s"),
    scratch_types=[
        pltpu.VMEM((CH, D), jnp.float32),
        pltpu.VMEM((CH, D), jnp.float32),
        pltpu.VMEM((CH, D), jnp.float32),
        pltpu.VMEM((CH, D), jnp.float32),
        pltpu.SemaphoreType.DMA,
        pltpu.SemaphoreType.DMA,
        pltpu.SemaphoreType.DMA,
        pltpu.SemaphoreType.DMA,
        pltpu.SemaphoreType.DMA,
        pltpu.SemaphoreType.DMA,
    ],
)(_pos_add_body)


TM = 512


def _tc_body(x_ref, pos_ref, o_ref):
    o_ref[...] = x_ref[...] + pos_ref[...]


_pos_add_tc = pl.pallas_call(
    _tc_body,
    grid=(B - SB, M // TM),
    in_specs=[
        pl.BlockSpec((1, TM, D), lambda b, i: (b + SB, i, 0)),
        pl.BlockSpec((TM, D), lambda b, i: (i, 0)),
    ],
    out_specs=pl.BlockSpec((1, TM, D), lambda b, i: (b, i, 0)),
    out_shape=jax.ShapeDtypeStruct((B - SB, M, D), jnp.float32),
)


@jax.jit
def kernel(x, pos_table):
    sc_out = _pos_add_sc(x.reshape(B * M, D), pos_table)
    tc_out = _pos_add_tc(x, pos_table)
    return jnp.concatenate([sc_out.reshape(SB, M, D), tc_out], axis=0)


# speedup vs baseline: 1.4580x; 1.4580x over previous
import functools

import jax
import jax.numpy as jnp
from jax import lax
from jax.experimental import pallas as pl
from jax.experimental.pallas import tpu as pltpu
from jax.experimental.pallas import tpu_sc as plsc

B = 4
M = 8192
D = 768
NC = 2
NS = 16
NW = NC * NS
POS_PER_W = M // NW
CH = 32
CHUNKS = POS_PER_W // CH
VPR = D // 16
NIT = CHUNKS * B
NBUF = 3


def _pos_add_body(x_hbm, pos_hbm, out_hbm,
                  xv0, xv1, xv2, pv0, pv1,
                  sin0, sin1, sin2, sout0, sout1, sout2, sp0, sp1):
    wid = lax.axis_index("s") * NC + lax.axis_index("c")
    row0 = wid * POS_PER_W
    xv = [xv0, xv1, xv2]
    pv = [pv0, pv1]
    sin = [sin0, sin1, sin2]
    sout = [sout0, sout1, sout2]
    sp = [sp0, sp1]

    def x_row(g):
        c, b = divmod(g, B)
        return b * M + row0 + c * CH

    in_h = [None] * NIT
    out_h = [None] * NIT
    pos_h = [None] * CHUNKS

    pos_h[0] = pltpu.async_copy(pos_hbm.at[pl.ds(row0, CH)], pv[0], sp[0])
    in_h[0] = pltpu.async_copy(x_hbm.at[pl.ds(x_row(0), CH)], xv[0], sin[0])
    in_h[1] = pltpu.async_copy(x_hbm.at[pl.ds(x_row(1), CH)], xv[1], sin[1])

    for g in range(NIT):
        s = g % NBUF
        c = g // B
        if g % B == 0 and c + 1 < CHUNKS:
            pos_h[c + 1] = pltpu.async_copy(
                pos_hbm.at[pl.ds(row0 + (c + 1) * CH, CH)],
                pv[(c + 1) % 2], sp[(c + 1) % 2])
        if g + 2 < NIT:
            if g >= 1:
                out_h[g - 1].wait()
            in_h[g + 2] = pltpu.async_copy(
                x_hbm.at[pl.ds(x_row(g + 2), CH)], xv[(g + 2) % NBUF],
                sin[(g + 2) % NBUF])
        if g % B == 0:
            pos_h[c].wait()
        in_h[g].wait()

        pvs = pv[c % 2]
        xvs = xv[s]

        @plsc.parallel_loop(0, CH, unroll=1)
        def _row_body(r):
            @plsc.parallel_loop(0, VPR, unroll=16)
            def _vec_body(v):
                j = v * 16
                plsc.addupdate(xvs.at[r, pl.ds(j, 16)], pvs[r, pl.ds(j, 16)])

        out_h[g] = pltpu.async_copy(xvs, out_hbm.at[pl.ds(x_row(g), CH)], sout[s])

    out_h[NIT - 2].wait()
    out_h[NIT - 1].wait()


_pos_add = functools.partial(
    pl.kernel,
    out_type=jax.ShapeDtypeStruct((B * M, D), jnp.float32),
    mesh=plsc.VectorSubcoreMesh(core_axis_name="c", subcore_axis_name="s"),
    scratch_types=[
        pltpu.VMEM((CH, D), jnp.float32),
        pltpu.VMEM((CH, D), jnp.float32),
        pltpu.VMEM((CH, D), jnp.float32),
        pltpu.VMEM((CH, D), jnp.float32),
        pltpu.VMEM((CH, D), jnp.float32),
        pltpu.SemaphoreType.DMA,
        pltpu.SemaphoreType.DMA,
        pltpu.SemaphoreType.DMA,
        pltpu.SemaphoreType.DMA,
        pltpu.SemaphoreType.DMA,
        pltpu.SemaphoreType.DMA,
        pltpu.SemaphoreType.DMA,
        pltpu.SemaphoreType.DMA,
    ],
)(_pos_add_body)


@jax.jit
def kernel(x, pos_table):
    out = _pos_add(x.reshape(B * M, D), pos_table)
    return out.reshape(x.shape)
